# Initial kernel scaffold; baseline (speedup 1.0000x reference)
import jax
import jax.numpy as jnp
from jax.experimental import pallas as pl


def kernel(x, pos, params, edge_index):
    raise NotImplementedError("write your pallas kernel here")


# speedup vs baseline: 2.5803x; 2.5803x over previous
import jax
import jax.numpy as jnp
from jax import lax
from jax.experimental import pallas as pl
from jax.experimental.pallas import tpu as pltpu
from jax.experimental.pallas import tpu_sc as plsc

_N = 10000
_E = 320000
_D = 128
_PW = 16
_NP = 10240
_EP = 327680
_BN = 2560
_BE = 2048
_NC = 2
_NS = 16
_NW = _NC * _NS
_EPW = _EP // _NW
_CH = 128
_NCH = _EPW // _CH
_RPT = _NP // _NS

_F32 = jnp.float32


def _lnorm(t, g, b):
    mu = jnp.mean(t, axis=1, keepdims=True)
    xc = t - mu
    var = jnp.mean(xc * xc, axis=1, keepdims=True)
    return xc / jnp.sqrt(var + 1e-5) * g + b


def _prep_body(h, an, bn, b1n, p_o, q_o):
    hv = h[...]
    p_o[...] = jnp.dot(hv, an[...], preferred_element_type=_F32) + b1n[...]
    q_o[...] = jnp.dot(hv, bn[...], preferred_element_type=_F32)


def _edge_body(g, pdv, wd, g1, be1, w2, b2, g2, be2,
               pw1, pb1, pg1, pbe1, pw2, pb2, msg_o, pmsg_o):
    pd = pdv[...]
    lane = lax.broadcasted_iota(jnp.int32, (1, _PW), 1)
    sumsq = jnp.sum(pd * pd, axis=1, keepdims=True)
    dist = jnp.sqrt(sumsq + 1e-12)
    dist_b = dist.astype(jnp.bfloat16).astype(_F32)
    wd_b = wd[...].astype(jnp.bfloat16).astype(_F32)
    t = g[...] + dist_b * wd_b
    a = jnp.maximum(_lnorm(t, g1[...], be1[...]), 0.0)
    m = jnp.dot(a, w2[...], preferred_element_type=_F32) + b2[...]
    m = jnp.maximum(_lnorm(m, g2[...], be2[...]), 0.0)
    msg_o[...] = m
    u = jnp.dot(m, pw1[...], preferred_element_type=_F32) + pb1[...]
    mu = jnp.sum(u, axis=1, keepdims=True) / 10.0
    uc = jnp.where(lane < 10, u - mu, 0.0)
    var = jnp.sum(uc * uc, axis=1, keepdims=True) / 10.0
    un = (u - mu) / jnp.sqrt(var + 1e-5) * pg1[...] + pbe1[...]
    un = jnp.maximum(un, 0.0)
    s = jnp.sum(un * pw2[...], axis=1, keepdims=True) + pb2[...]
    pmsg_o[...] = jnp.where(lane < 3, pd * s, (lane == 3).astype(_F32))


def _node_body(h, pos, m0, m1, p0, p1, w1, b1, g1, be1, w2, b2, g2, be2,
               an, bn, b1n, h_o, pos_o, pp_o, qq_o):
    magg = m0[...] + m1[...]
    prow = p0[...] + p1[...]
    lane = lax.broadcasted_iota(jnp.int32, (1, _PW), 1)
    cnt = jnp.sum(prow * (lane == 3).astype(_F32), axis=1, keepdims=True)
    pagg = prow / jnp.maximum(cnt, 1.0) * (lane < 3).astype(_F32)
    pos_o[...] = pos[...] + pagg
    w1v = w1[...]
    u = (jnp.dot(h[...], w1v[:_D], preferred_element_type=_F32)
         + jnp.dot(magg, w1v[_D:], preferred_element_type=_F32) + b1[...])
    u = jnp.maximum(_lnorm(u, g1[...], be1[...]), 0.0)
    u = jnp.dot(u, w2[...], preferred_element_type=_F32) + b2[...]
    u = jnp.maximum(_lnorm(u, g2[...], be2[...]), 0.0)
    hn = h[...] + u
    h_o[...] = hn
    pp_o[...] = jnp.dot(hn, an[...], preferred_element_type=_F32) + b1n[...]
    qq_o[...] = jnp.dot(hn, bn[...], preferred_element_type=_F32)


def _full(shape):
    return pl.BlockSpec(shape, lambda i: tuple(0 for _ in shape))


_nbs = pl.BlockSpec((_BN, _D), lambda i: (i, 0))
_nps = pl.BlockSpec((_BN, _PW), lambda i: (i, 0))
_ebs = pl.BlockSpec((_BE, _D), lambda i: (i, 0))
_eps = pl.BlockSpec((_BE, _PW), lambda i: (i, 0))
_NBLK = _NP // _BN


def _prep_call(hT, an, bn, b1n, interpret=False):
    return pl.pallas_call(
        _prep_body,
        grid=(_NBLK,),
        in_specs=[_nbs, _full((_D, _D)), _full((_D, _D)), _full((1, _D))],
        out_specs=[_nbs, _nbs],
        out_shape=[jax.ShapeDtypeStruct((_NP, _D), _F32)] * 2,
        interpret=interpret,
    )(hT, an, bn, b1n)


def _edge_call(g, pdv, w, interpret=False):
    wspecs = ([_full((1, _D))] * 3 + [_full((_D, _D))] + [_full((1, _D))] * 3
              + [_full((_D, _PW))] + [_full((1, _PW))] * 4 + [_full((1, 1))])
    return pl.pallas_call(
        _edge_body,
        grid=(_EP // _BE,),
        in_specs=[_ebs, _eps] + wspecs,
        out_specs=[_ebs, _eps],
        out_shape=[jax.ShapeDtypeStruct((_EP, _D), _F32),
                   jax.ShapeDtypeStruct((_EP, _PW), _F32)],
        interpret=interpret,
    )(g, pdv, w['wd'], w['mg1'], w['mbe1'], w['mw2'], w['mb2'],
      w['mg2'], w['mbe2'], w['pw1'], w['pb1'], w['pg1'], w['pbe1'],
      w['pw2'], w['pb2'])


def _node_call(hT, posT, magg, pagg, w, wn, interpret=False):
    m0 = pl.BlockSpec((_BN, _D), lambda i: (i, 0))
    m1 = pl.BlockSpec((_BN, _D), lambda i: (i + _NBLK, 0))
    p0 = pl.BlockSpec((_BN, _PW), lambda i: (i, 0))
    p1 = pl.BlockSpec((_BN, _PW), lambda i: (i + _NBLK, 0))
    wspecs = ([_full((2 * _D, _D))] + [_full((1, _D))] * 3
              + [_full((_D, _D))] + [_full((1, _D))] * 3
              + [_full((_D, _D))] * 2 + [_full((1, _D))])
    return pl.pallas_call(
        _node_body,
        grid=(_NBLK,),
        in_specs=[_nbs, _nps, m0, m1, p0, p1] + wspecs,
        out_specs=[_nbs, _nps, _nbs, _nbs],
        out_shape=[jax.ShapeDtypeStruct((_NP, _D), _F32),
                   jax.ShapeDtypeStruct((_NP, _PW), _F32),
                   jax.ShapeDtypeStruct((_NP, _D), _F32),
                   jax.ShapeDtypeStruct((_NP, _D), _F32)],
        interpret=interpret,
    )(hT, posT, magg, magg, pagg, pagg, w['uw1'], w['ub1'], w['ug1'],
      w['ube1'], w['uw2'], w['ub2'], w['ug2'], w['ube2'],
      wn['A'], wn['B'], wn['b1'])


_mesh_cache = []


def _mesh():
    if not _mesh_cache:
        _mesh_cache.append(plsc.VectorSubcoreMesh(
            core_axis_name="c", subcore_axis_name="s",
            num_cores=_NC, num_subcores=_NS))
    return _mesh_cache[0]


def _gather_body(p_hbm, q_hbm, pos_hbm, src_hbm, dst_hbm, g_hbm, pd_hbm,
                 idxs0, idxd0, bufp0, bufq0, bufs0, bufd0,
                 idxs1, idxd1, bufp1, bufq1, bufs1, bufd1,
                 gsem0, gsem1, isem0, isem1, ssem0, ssem1):
    cid = lax.axis_index("c")
    sid = lax.axis_index("s")
    wid = sid * _NC + cid
    base = wid * _EPW
    sets = ((idxs0, idxd0, bufp0, bufq0, bufs0, bufd0, gsem0, isem0, ssem0),
            (idxs1, idxd1, bufp1, bufq1, bufs1, bufd1, gsem1, isem1, ssem1))

    def idx_issue(i, st):
        idxs, idxd = st[0], st[1]
        isem = st[7]
        b = base + i * _CH
        pltpu.async_copy(src_hbm.at[pl.ds(b, _CH)], idxs, isem)
        pltpu.async_copy(dst_hbm.at[pl.ds(b, _CH)], idxd, isem)

    def idx_drain(st):
        idxs, idxd = st[0], st[1]
        isem = st[7]
        pltpu.make_async_copy(src_hbm.at[pl.ds(0, _CH)], idxs, isem).wait()
        pltpu.make_async_copy(dst_hbm.at[pl.ds(0, _CH)], idxd, isem).wait()

    def g_issue(st):
        idxs, idxd, bufp, bufq, bufs, bufd = st[:6]
        gsem = st[6]
        pltpu.async_copy(p_hbm.at[idxd], bufp, gsem)
        pltpu.async_copy(q_hbm.at[idxs], bufq, gsem)
        pltpu.async_copy(pos_hbm.at[idxs], bufs, gsem)
        pltpu.async_copy(pos_hbm.at[idxd], bufd, gsem)

    def g_drain(st):
        bufp, bufq, bufs, bufd = st[2:6]
        gsem = st[6]
        pltpu.make_async_copy(p_hbm.at[pl.ds(0, _CH)], bufp, gsem).wait()
        pltpu.make_async_copy(q_hbm.at[pl.ds(0, _CH)], bufq, gsem).wait()
        pltpu.make_async_copy(pos_hbm.at[pl.ds(0, _CH)], bufs, gsem).wait()
        pltpu.make_async_copy(pos_hbm.at[pl.ds(0, _CH)], bufd, gsem).wait()

    def s_drain(st):
        bufp, bufs = st[2], st[4]
        ssem = st[8]
        pltpu.make_async_copy(bufp, g_hbm.at[pl.ds(0, _CH)], ssem).wait()
        pltpu.make_async_copy(bufs, pd_hbm.at[pl.ds(0, _CH)], ssem).wait()

    def process(i, st):
        bufp, bufq, bufs, bufd = st[2:6]
        ssem = st[8]
        b = base + i * _CH

        def row(e, c):
            for j in range(_D // 16):
                sl = pl.ds(j * 16, 16)
                bufp[e, sl] = bufp[e, sl] + bufq[e, sl]
            bufs[e, :] = bufd[e, :] - bufs[e, :]
            return c

        lax.fori_loop(0, _CH, row, 0)
        pltpu.async_copy(bufp, g_hbm.at[pl.ds(b, _CH)], ssem)
        pltpu.async_copy(bufs, pd_hbm.at[pl.ds(b, _CH)], ssem)

    pltpu.sync_copy(src_hbm.at[pl.ds(base, _CH)], idxs0)
    pltpu.sync_copy(dst_hbm.at[pl.ds(base, _CH)], idxd0)
    g_issue(sets[0])
    idx_issue(1, sets[1])

    def outer(k, c):
        idx_drain(sets[1])

        @pl.when(k >= 1)
        def _():
            s_drain(sets[1])

        g_issue(sets[1])
        g_drain(sets[0])
        process(2 * k, sets[0])

        @pl.when(2 * k + 2 < _NCH)
        def _():
            idx_issue(2 * k + 2, sets[0])

        g_drain(sets[1])
        process(2 * k + 1, sets[1])

        @pl.when(2 * k + 2 < _NCH)
        def _():
            s_drain(sets[0])
            idx_drain(sets[0])
            g_issue(sets[0])

        @pl.when(2 * k + 3 < _NCH)
        def _():
            idx_issue(2 * k + 3, sets[1])

        return c

    lax.fori_loop(0, _NCH // 2, outer, 0)
    s_drain(sets[0])
    s_drain(sets[1])


def _gather_call(P, Q, posT, srcp, dstp):
    fn = pl.kernel(
        _gather_body,
        out_type=[jax.ShapeDtypeStruct((_EP, _D), _F32),
                  jax.ShapeDtypeStruct((_EP, _PW), _F32)],
        mesh=_mesh(),
        scratch_types=[
            pltpu.VMEM((_CH,), jnp.int32),
            pltpu.VMEM((_CH,), jnp.int32),
            pltpu.VMEM((_CH, _D), _F32),
            pltpu.VMEM((_CH, _D), _F32),
            pltpu.VMEM((_CH, _PW), _F32),
            pltpu.VMEM((_CH, _PW), _F32),
            pltpu.VMEM((_CH,), jnp.int32),
            pltpu.VMEM((_CH,), jnp.int32),
            pltpu.VMEM((_CH, _D), _F32),
            pltpu.VMEM((_CH, _D), _F32),
            pltpu.VMEM((_CH, _PW), _F32),
            pltpu.VMEM((_CH, _PW), _F32),
            pltpu.SemaphoreType.DMA,
            pltpu.SemaphoreType.DMA,
            pltpu.SemaphoreType.DMA,
            pltpu.SemaphoreType.DMA,
            pltpu.SemaphoreType.DMA,
            pltpu.SemaphoreType.DMA,
        ],
        compiler_params=pltpu.CompilerParams(use_tc_tiling_on_sc=False),
    )
    return fn(P, Q, posT, srcp, dstp)


def _scatter_body(msg_hbm, pmsg_hbm, dst_hbm, magg_hbm, pagg_hbm,
                  macc_sh, pacc_sh, idx0, bufm0, bufpm0,
                  idx1, bufm1, bufpm1, rsem0, rsem1):
    cid = lax.axis_index("c")
    sid = lax.axis_index("s")
    wid = sid * _NC + cid
    base = wid * _EPW
    r0 = sid * _RPT
    zrow = jnp.zeros((16,), _F32)

    def zb_loop(i, c):
        for j in range(_D // 16):
            bufm0[i, pl.ds(j * 16, 16)] = zrow
        bufpm0[i, :] = zrow
        return c

    lax.fori_loop(0, _CH, zb_loop, 0)
    for k in range(_RPT // _CH):
        pltpu.sync_copy(bufm0, macc_sh.at[pl.ds(r0 + k * _CH, _CH)])
        pltpu.sync_copy(bufpm0, pacc_sh.at[pl.ds(r0 + k * _CH, _CH)])
    plsc.subcore_barrier()

    sets = ((idx0, bufm0, bufpm0, rsem0), (idx1, bufm1, bufpm1, rsem1))

    def r_issue(i, st):
        idx, bufm, bufpm, rsem = st
        b = base + i * _CH
        pltpu.async_copy(dst_hbm.at[pl.ds(b, _CH)], idx, rsem)
        pltpu.async_copy(msg_hbm.at[pl.ds(b, _CH)], bufm, rsem)
        pltpu.async_copy(pmsg_hbm.at[pl.ds(b, _CH)], bufpm, rsem)

    def r_drain(st):
        idx, bufm, bufpm, rsem = st
        pltpu.make_async_copy(dst_hbm.at[pl.ds(0, _CH)], idx, rsem).wait()
        pltpu.make_async_copy(msg_hbm.at[pl.ds(0, _CH)], bufm, rsem).wait()
        pltpu.make_async_copy(pmsg_hbm.at[pl.ds(0, _CH)], bufpm, rsem).wait()

    def addc(st):
        idx, bufm, bufpm, rsem = st
        pltpu.sync_copy(bufm, macc_sh.at[idx], add=True)
        pltpu.sync_copy(bufpm, pacc_sh.at[idx], add=True)

    r_issue(0, sets[0])

    def chunk2(k, c):
        for s in (0, 1):
            i = 2 * k + s

            @pl.when(i + 1 < _NCH)
            def _():
                r_issue(i + 1, sets[1 - s])

            r_drain(sets[s])
            addc(sets[s])
        return c

    lax.fori_loop(0, _NCH // 2, chunk2, 0)
    plsc.subcore_barrier()
    for k in range(_RPT // _CH):
        bm = (bufm0, bufm1)[k % 2]
        bp = (bufpm0, bufpm1)[k % 2]
        pltpu.sync_copy(macc_sh.at[pl.ds(r0 + k * _CH, _CH)], bm)
        pltpu.sync_copy(bm, magg_hbm.at[pl.ds(cid * _NP + r0 + k * _CH, _CH)])
        pltpu.sync_copy(pacc_sh.at[pl.ds(r0 + k * _CH, _CH)], bp)
        pltpu.sync_copy(bp, pagg_hbm.at[pl.ds(cid * _NP + r0 + k * _CH, _CH)])


def _scatter_call(msg, pmsg, dstp):
    fn = pl.kernel(
        _scatter_body,
        out_type=[jax.ShapeDtypeStruct((_NC * _NP, _D), _F32),
                  jax.ShapeDtypeStruct((_NC * _NP, _PW), _F32)],
        mesh=_mesh(),
        scratch_types=[
            pltpu.VMEM_SHARED((_NP, _D), _F32),
            pltpu.VMEM_SHARED((_NP, _PW), _F32),
            pltpu.VMEM((_CH,), jnp.int32),
            pltpu.VMEM((_CH, _D), _F32),
            pltpu.VMEM((_CH, _PW), _F32),
            pltpu.VMEM((_CH,), jnp.int32),
            pltpu.VMEM((_CH, _D), _F32),
            pltpu.VMEM((_CH, _PW), _F32),
            pltpu.SemaphoreType.DMA,
            pltpu.SemaphoreType.DMA,
        ],
        compiler_params=pltpu.CompilerParams(use_tc_tiling_on_sc=False),
    )
    return fn(msg, pmsg, dstp)


def _prep_weights(p):
    w = {}
    w['A'] = p['msg_w1'][:_D]
    w['B'] = p['msg_w1'][_D:2 * _D]
    w['wd'] = p['msg_w1'][2 * _D:2 * _D + 1]
    w['b1'] = p['msg_b1'][None]
    w['mg1'] = p['msg_g1'][None]
    w['mbe1'] = p['msg_be1'][None]
    w['mw2'] = p['msg_w2']
    w['mb2'] = p['msg_b2'][None]
    w['mg2'] = p['msg_g2'][None]
    w['mbe2'] = p['msg_be2'][None]
    w['pw1'] = jnp.pad(p['pos_w1'], ((0, 0), (0, _PW - 10)))
    w['pb1'] = jnp.pad(p['pos_b1'], (0, _PW - 10))[None]
    w['pg1'] = jnp.pad(p['pos_g1'], (0, _PW - 10))[None]
    w['pbe1'] = jnp.pad(p['pos_be1'], (0, _PW - 10))[None]
    w['pw2'] = jnp.pad(p['pos_w2'][:, 0], (0, _PW - 10))[None]
    w['pb2'] = p['pos_b2'].reshape(1, 1)
    w['uw1'] = p['upd_w1']
    w['ub1'] = p['upd_b1'][None]
    w['ug1'] = p['upd_g1'][None]
    w['ube1'] = p['upd_be1'][None]
    w['uw2'] = p['upd_w2']
    w['ub2'] = p['upd_b2'][None]
    w['ug2'] = p['upd_g2'][None]
    w['ube2'] = p['upd_be2'][None]
    return w


def kernel(x, pos, params, edge_index):
    hT = jnp.pad(x, ((0, _NP - _N), (0, 0)))
    posT = jnp.pad(pos, ((0, _NP - _N), (0, _PW - 3)))
    srcp = jnp.pad(edge_index[0], (0, _EP - _E), constant_values=_N)
    dstp = jnp.pad(edge_index[1], (0, _EP - _E), constant_values=_N)
    prm = [_prep_weights(p) for p in params]
    P, Q = _prep_call(hT, prm[0]['A'], prm[0]['B'], prm[0]['b1'])
    nl = len(params)
    for l in range(nl):
        w = prm[l]
        wn = prm[(l + 1) % nl]
        g, pdv = _gather_call(P, Q, posT, srcp, dstp)
        msg, pmsg = _edge_call(g, pdv, w)
        magg, pagg = _scatter_call(msg, pmsg, dstp)
        hT, posT, P, Q = _node_call(hT, posT, magg, pagg, w, wn)
    return (posT[:_N, :3], hT[:_N])
